# baseline (speedup 1.0000x reference)
import functools

import jax
import jax.numpy as jnp
from jax import lax
from jax.experimental import pallas as pl
from jax.experimental.pallas import tpu as pltpu
from jax.experimental.pallas import tpu_sc as plsc

N = 10000
E = 320000
D = 128
R = 4
VOCAB = 100
B = 256

NS = 16
SLAB = 32
NQ = 4
CHUNK = 128
ECH = 2560
E_PAD = ECH * CHUNK
CPT = ECH // NS
SEG = 32
NBUF = 8
ACC_R = 40960
RPT = ACC_R // NS
NH = 10240
BN = 1000
NB = N // BN

_mesh = lambda: plsc.VectorSubcoreMesh(core_axis_name="c", subcore_axis_name="s")


def _sc_prep(e4, atom4, dstadj2, ones_in, zsm,
             h0m, cnt_out,
             cnt_sh, aidx, didx, rows, ones_v, sem):
  c = lax.axis_index("c")
  s = lax.axis_index("s")

  @pl.when(c == 1)
  def _():
    part = s % 4
    q = s // 4
    pltpu.sync_copy(atom4.at[s], aidx)

    def chunk(k, carry):
      base = part * 2560 + k * CHUNK
      pltpu.async_copy(e4.at[aidx.at[k]], rows, sem).wait()
      pltpu.sync_copy(rows, h0m.at[pl.ds(base, CHUNK), pl.ds(32 * q, SLAB)])
      return carry
    lax.fori_loop(0, 20, chunk, 0)

  @pl.when(c == 0)
  def _():
    pltpu.sync_copy(ones_in, ones_v)

    def zinit(k, carry):
      pltpu.sync_copy(zsm, cnt_sh.at[pl.ds(s * RPT + k * CHUNK, CHUNK)])
      return carry
    lax.fori_loop(0, RPT // CHUNK, zinit, 0)
    pltpu.sync_copy(dstadj2.at[s], didx)
    plsc.subcore_barrier()

    def chunk(k, carry):
      pltpu.sync_copy(ones_v, cnt_sh.at[didx.at[k]], add=True)
      return carry
    lax.fori_loop(0, CPT, chunk, 0)
    plsc.subcore_barrier()
    pltpu.sync_copy(cnt_sh.at[pl.ds(s * RPT, RPT)], cnt_out.at[pl.ds(s * RPT, RPT)])


def _sc_edge(hv, src40, src41, src42, src43, dstadj2, zsm,
             aout,
             acc_sh, sidx, didx, rows, gsems, ssems):
  c = lax.axis_index("c")
  s = lax.axis_index("s")
  src4 = (src40, src41, src42, src43)

  for p in (0, 1):
    for cc in (0, 1):
      @pl.when(c == cc)
      def _(q=2 * cc + p):
        def zinit(k, carry):
          pltpu.sync_copy(zsm, acc_sh.at[pl.ds(s * RPT + k * CHUNK, CHUNK)])
          return carry
        lax.fori_loop(0, RPT // CHUNK, zinit, 0)
        plsc.subcore_barrier()

        for seg in range(CPT // SEG):
          pltpu.sync_copy(src4[q].at[s].at[pl.ds(seg * SEG, SEG)], sidx)
          pltpu.sync_copy(dstadj2.at[s].at[pl.ds(seg * SEG, SEG)], didx)

          def octet(j, carry):
            k0 = NBUF * j
            gs = [pltpu.async_copy(hv.at[sidx.at[k0 + u]], rows[u], gsems[u])
                  for u in range(NBUF)]
            ss = []
            for u in range(NBUF):
              gs[u].wait()
              ss.append(pltpu.async_copy(rows[u], acc_sh.at[didx.at[k0 + u]],
                                         ssems[u], add=True))
            for u in range(NBUF):
              ss[u].wait()
            return carry
          lax.fori_loop(0, SEG // NBUF, octet, 0)
        plsc.subcore_barrier()
        pltpu.sync_copy(acc_sh.at[pl.ds(s * RPT, RPT)],
                        aout.at[pl.ds(s * RPT, RPT), pl.ds(32 * q, SLAB)])


def _sc_prep_call(e4, atom4, dstadj2):
  f32 = jnp.float32
  ones_in = jnp.ones((CHUNK, 16), f32)
  zsm = jnp.zeros((CHUNK, 16), f32)
  fn = pl.kernel(
      _sc_prep,
      out_type=[jax.ShapeDtypeStruct((NH, D), f32),
                jax.ShapeDtypeStruct((ACC_R, 16), f32)],
      mesh=_mesh(),
      compiler_params=pltpu.CompilerParams(use_tc_tiling_on_sc=False),
      scratch_types=[
          pltpu.VMEM_SHARED((ACC_R, 16), f32),
          pltpu.VMEM((20, CHUNK), jnp.int32),
          pltpu.VMEM((CPT, CHUNK), jnp.int32),
          pltpu.VMEM((CHUNK, SLAB), f32),
          pltpu.VMEM((CHUNK, 16), f32),
          pltpu.SemaphoreType.DMA,
      ],
  )
  return fn(e4, atom4, dstadj2, ones_in, zsm)


def _sc_edge_call(hv, src4, dstadj2):
  f32 = jnp.float32
  zsm = jnp.zeros((CHUNK, SLAB), f32)
  fn = pl.kernel(
      _sc_edge,
      out_type=jax.ShapeDtypeStruct((ACC_R, D), f32),
      mesh=_mesh(),
      compiler_params=pltpu.CompilerParams(use_tc_tiling_on_sc=False),
      scratch_types=[
          pltpu.VMEM_SHARED((ACC_R, SLAB), f32),
          pltpu.VMEM((SEG, CHUNK), jnp.int32),
          pltpu.VMEM((SEG, CHUNK), jnp.int32),
          [pltpu.VMEM((CHUNK, SLAB), f32) for _ in range(NBUF)],
          [pltpu.SemaphoreType.DMA for _ in range(NBUF)],
          [pltpu.SemaphoreType.DMA for _ in range(NBUF)],
      ],
  )
  return fn(hv, *src4, dstadj2, zsm)


def _tc_layer0(h, a0, a1, a2, a3, c0, c1, c2, c3, w, root, b, out):
  accs = (a0, a1, a2, a3)
  cnts = (c0, c1, c2, c3)
  res = jnp.dot(h[...], root[...], preferred_element_type=jnp.float32) + b[...]
  for r in range(R):
    inv = 1.0 / jnp.maximum(cnts[r][:, 0:1], 1.0)
    res = res + jnp.dot(accs[r][...] * inv, w[r],
                        preferred_element_type=jnp.float32)
  out[...] = jnp.tanh(res)


def _tc_layer1(h, a0, a1, a2, a3, c0, c1, c2, c3, w, root, b, batch2,
               final, psum_s, pcnt_s):
  accs = (a0, a1, a2, a3)
  cnts = (c0, c1, c2, c3)
  i = pl.program_id(0)
  res = jnp.dot(h[...], root[...], preferred_element_type=jnp.float32) + b[...]
  for r in range(R):
    inv = 1.0 / jnp.maximum(cnts[r][:, 0:1], 1.0)
    res = res + jnp.dot(accs[r][...] * inv, w[r],
                        preferred_element_type=jnp.float32)

  oh = (lax.broadcasted_iota(jnp.int32, (B, BN), 0) == batch2[0]).astype(jnp.float32)
  rsum = jnp.dot(res, jnp.ones((D, 1), jnp.float32), preferred_element_type=jnp.float32)
  pv = jnp.dot(oh, rsum, preferred_element_type=jnp.float32)
  pc = jnp.dot(oh, jnp.ones((BN, 1), jnp.float32), preferred_element_type=jnp.float32)

  @pl.when(i == 0)
  def _():
    psum_s[...] = pv
    pcnt_s[...] = pc

  @pl.when(i != 0)
  def _():
    psum_s[...] = psum_s[...] + pv
    pcnt_s[...] = pcnt_s[...] + pc

  @pl.when(i == NB - 1)
  def _():
    final[...] = psum_s[...] / (jnp.float32(D) * jnp.maximum(pcnt_s[...], 1.0))


def _tc_layer_call(hm, accm, cnt, w, root, b, last, batch2=None):
  f32 = jnp.float32
  h_spec = pl.BlockSpec((BN, D), lambda i: (i, 0))
  a_spec = [pl.BlockSpec((BN, D), lambda i, r=r: (r * NB + i, 0))
            for r in range(R)]
  cnt_spec = [pl.BlockSpec((BN, 16), lambda i, r=r: (r * NB + i, 0))
              for r in range(R)]
  w_spec = pl.BlockSpec((R, D, D), lambda i: (0, 0, 0))
  root_spec = pl.BlockSpec((D, D), lambda i: (0, 0))
  b_spec = pl.BlockSpec((1, D), lambda i: (0, 0))
  params = pltpu.CompilerParams(dimension_semantics=("arbitrary",))
  if not last:
    return pl.pallas_call(
        _tc_layer0,
        grid=(NB,),
        in_specs=[h_spec] + a_spec + cnt_spec + [w_spec, root_spec, b_spec],
        out_specs=pl.BlockSpec((BN, D), lambda i: (i, 0)),
        out_shape=jax.ShapeDtypeStruct((N, D), f32),
        compiler_params=params,
    )(hm, *([accm] * R), *([cnt] * R), w, root, b)
  batch_spec = pl.BlockSpec((1, 1, BN), lambda i: (i, 0, 0))
  return pl.pallas_call(
      _tc_layer1,
      grid=(NB,),
      in_specs=[h_spec] + a_spec + cnt_spec + [w_spec, root_spec, b_spec, batch_spec],
      out_specs=pl.BlockSpec((B, 1), lambda i: (0, 0)),
      out_shape=jax.ShapeDtypeStruct((B, 1), f32),
      scratch_shapes=[pltpu.VMEM((B, 1), f32), pltpu.VMEM((B, 1), f32)],
      compiler_params=params,
  )(hm, *([accm] * R), *([cnt] * R), w, root, b, batch2)


def kernel(atom_type, edge_index, edge_type, batch, emb, W0, root0, b0, W1, root1, b1):
  i32 = jnp.int32
  src = edge_index[0].astype(i32)
  dst = edge_index[1].astype(i32)
  et = edge_type.astype(i32)

  pad_e = E_PAD - E
  ar = jnp.arange(pad_e, dtype=i32)
  srcp = jnp.concatenate([src, ar % N])
  src4 = tuple((4 * srcp + q).reshape(NS, CPT, CHUNK) for q in range(NQ))
  dstadj2 = jnp.concatenate(
      [et * N + dst, R * N + (ar % (ACC_R - R * N))]).reshape(NS, CPT, CHUNK)
  atomp = jnp.concatenate(
      [atom_type.astype(i32), jnp.arange(NH - N, dtype=i32) % VOCAB])
  atom4 = jnp.stack([4 * atomp + q for q in range(NQ)]).reshape(NQ * 4, 20, CHUNK)
  batch2 = batch.astype(i32).reshape(NB, 1, BN)

  e4 = emb.reshape(4 * VOCAB, SLAB)

  h0m, cnt = _sc_prep_call(e4, atom4, dstadj2)
  h0v = h0m.reshape(4 * NH, SLAB)
  acc0 = _sc_edge_call(h0v, src4, dstadj2)
  h1m = _tc_layer_call(h0m, acc0, cnt, W0, root0, b0.reshape(1, D), last=False)
  h1v = h1m.reshape(4 * N, SLAB)
  acc1 = _sc_edge_call(h1v, src4, dstadj2)
  final = _tc_layer_call(h1m, acc1, cnt, W1, root1, b1.reshape(1, D),
                         last=True, batch2=batch2)
  return final[:, 0]


# speedup vs baseline: 1.2121x; 1.2121x over previous
import functools

import jax
import jax.numpy as jnp
from jax import lax
from jax.experimental import pallas as pl
from jax.experimental.pallas import tpu as pltpu
from jax.experimental.pallas import tpu_sc as plsc

N = 10000
E = 320000
D = 128
R = 4
VOCAB = 100
B = 256

NS = 16
SLAB = 32
NQ = 4
CHUNK = 128
ECH = 2560
E_PAD = ECH * CHUNK
CPT = ECH // NS
SEG = 32
NBUF = 8
ACC_R = 40960
RPT = ACC_R // NS
NH = 10240
BN = 1000
NB = N // BN

_mesh = lambda: plsc.VectorSubcoreMesh(core_axis_name="c", subcore_axis_name="s")


def _sc_prep(e4, atom4, dstadj2, ones_in, zcnt,
             h0m, cnt_out,
             cnt_sh, aidx, didx, rows, ones_v, sem):
  c = lax.axis_index("c")
  s = lax.axis_index("s")

  @pl.when(c == 1)
  def _():
    part = s % 4
    q = s // 4
    pltpu.sync_copy(atom4.at[s], aidx)

    def chunk(k, carry):
      base = part * 2560 + k * CHUNK
      pltpu.async_copy(e4.at[aidx.at[k]], rows, sem).wait()
      pltpu.sync_copy(rows, h0m.at[pl.ds(base, CHUNK), pl.ds(32 * q, SLAB)])
      return carry
    lax.fori_loop(0, 20, chunk, 0)

  @pl.when(c == 0)
  def _():
    pltpu.sync_copy(ones_in, ones_v)
    pltpu.sync_copy(zcnt.at[pl.ds(s * RPT, RPT)], cnt_sh.at[pl.ds(s * RPT, RPT)])
    pltpu.sync_copy(dstadj2.at[s], didx)
    plsc.subcore_barrier()

    def chunk(k, carry):
      pltpu.sync_copy(ones_v, cnt_sh.at[didx.at[k]], add=True)
      return carry
    lax.fori_loop(0, CPT, chunk, 0)
    plsc.subcore_barrier()
    pltpu.sync_copy(cnt_sh.at[pl.ds(s * RPT, RPT)], cnt_out.at[pl.ds(s * RPT, RPT)])


def _sc_edge(hv, src40, src41, src42, src43, dstadj2, zacc,
             aout,
             acc_sh, sidx, didx, rows, gsems, ssems):
  c = lax.axis_index("c")
  s = lax.axis_index("s")
  src4 = (src40, src41, src42, src43)

  for p in (0, 1):
    for cc in (0, 1):
      @pl.when(c == cc)
      def _(q=2 * cc + p):
        pltpu.sync_copy(zacc.at[pl.ds(s * RPT, RPT)], acc_sh.at[pl.ds(s * RPT, RPT)])
        plsc.subcore_barrier()

        for seg in range(CPT // SEG):
          pltpu.sync_copy(src4[q].at[s].at[pl.ds(seg * SEG, SEG)], sidx)
          pltpu.sync_copy(dstadj2.at[s].at[pl.ds(seg * SEG, SEG)], didx)

          def octet(j, carry):
            k0 = NBUF * j
            gs = [pltpu.async_copy(hv.at[sidx.at[k0 + u]], rows[u], gsems[u])
                  for u in range(NBUF)]
            ss = []
            for u in range(NBUF):
              gs[u].wait()
              ss.append(pltpu.async_copy(rows[u], acc_sh.at[didx.at[k0 + u]],
                                         ssems[u], add=True))
            for u in range(NBUF):
              ss[u].wait()
            return carry
          lax.fori_loop(0, SEG // NBUF, octet, 0)
        plsc.subcore_barrier()
        pltpu.sync_copy(acc_sh.at[pl.ds(s * RPT, RPT)],
                        aout.at[pl.ds(s * RPT, RPT), pl.ds(32 * q, SLAB)])


def _sc_prep_call(e4, atom4, dstadj2):
  f32 = jnp.float32
  ones_in = jnp.ones((CHUNK, 16), f32)
  zcnt = jnp.zeros((ACC_R, 16), f32)
  fn = pl.kernel(
      _sc_prep,
      out_type=[jax.ShapeDtypeStruct((NH, D), f32),
                jax.ShapeDtypeStruct((ACC_R, 16), f32)],
      mesh=_mesh(),
      compiler_params=pltpu.CompilerParams(use_tc_tiling_on_sc=False),
      scratch_types=[
          pltpu.VMEM_SHARED((ACC_R, 16), f32),
          pltpu.VMEM((20, CHUNK), jnp.int32),
          pltpu.VMEM((CPT, CHUNK), jnp.int32),
          pltpu.VMEM((CHUNK, SLAB), f32),
          pltpu.VMEM((CHUNK, 16), f32),
          pltpu.SemaphoreType.DMA,
      ],
  )
  return fn(e4, atom4, dstadj2, ones_in, zcnt)


def _sc_edge_call(hv, src4, dstadj2):
  f32 = jnp.float32
  zacc = jnp.zeros((ACC_R, SLAB), f32)
  fn = pl.kernel(
      _sc_edge,
      out_type=jax.ShapeDtypeStruct((ACC_R, D), f32),
      mesh=_mesh(),
      compiler_params=pltpu.CompilerParams(use_tc_tiling_on_sc=False),
      scratch_types=[
          pltpu.VMEM_SHARED((ACC_R, SLAB), f32),
          pltpu.VMEM((SEG, CHUNK), jnp.int32),
          pltpu.VMEM((SEG, CHUNK), jnp.int32),
          [pltpu.VMEM((CHUNK, SLAB), f32) for _ in range(NBUF)],
          [pltpu.SemaphoreType.DMA for _ in range(NBUF)],
          [pltpu.SemaphoreType.DMA for _ in range(NBUF)],
      ],
  )
  return fn(hv, *src4, dstadj2, zacc)


def _tc_layer0(h, a0, a1, a2, a3, c0, c1, c2, c3, w, root, b, out):
  accs = (a0, a1, a2, a3)
  cnts = (c0, c1, c2, c3)
  res = jnp.dot(h[...], root[...], preferred_element_type=jnp.float32) + b[...]
  for r in range(R):
    inv = 1.0 / jnp.maximum(cnts[r][:, 0:1], 1.0)
    res = res + jnp.dot(accs[r][...] * inv, w[r],
                        preferred_element_type=jnp.float32)
  out[...] = jnp.tanh(res)


def _tc_layer1(h, a0, a1, a2, a3, c0, c1, c2, c3, w, root, b, batch2,
               final, psum_s, pcnt_s):
  accs = (a0, a1, a2, a3)
  cnts = (c0, c1, c2, c3)
  i = pl.program_id(0)
  res = jnp.dot(h[...], root[...], preferred_element_type=jnp.float32) + b[...]
  for r in range(R):
    inv = 1.0 / jnp.maximum(cnts[r][:, 0:1], 1.0)
    res = res + jnp.dot(accs[r][...] * inv, w[r],
                        preferred_element_type=jnp.float32)

  oh = (lax.broadcasted_iota(jnp.int32, (B, BN), 0) == batch2[0]).astype(jnp.float32)
  rsum = jnp.dot(res, jnp.ones((D, 1), jnp.float32), preferred_element_type=jnp.float32)
  pv = jnp.dot(oh, rsum, preferred_element_type=jnp.float32)
  pc = jnp.dot(oh, jnp.ones((BN, 1), jnp.float32), preferred_element_type=jnp.float32)

  @pl.when(i == 0)
  def _():
    psum_s[...] = pv
    pcnt_s[...] = pc

  @pl.when(i != 0)
  def _():
    psum_s[...] = psum_s[...] + pv
    pcnt_s[...] = pcnt_s[...] + pc

  @pl.when(i == NB - 1)
  def _():
    final[...] = psum_s[...] / (jnp.float32(D) * jnp.maximum(pcnt_s[...], 1.0))


def _tc_layer_call(hm, accm, cnt, w, root, b, last, batch2=None):
  f32 = jnp.float32
  h_spec = pl.BlockSpec((BN, D), lambda i: (i, 0))
  a_spec = [pl.BlockSpec((BN, D), lambda i, r=r: (r * NB + i, 0))
            for r in range(R)]
  cnt_spec = [pl.BlockSpec((BN, 16), lambda i, r=r: (r * NB + i, 0))
              for r in range(R)]
  w_spec = pl.BlockSpec((R, D, D), lambda i: (0, 0, 0))
  root_spec = pl.BlockSpec((D, D), lambda i: (0, 0))
  b_spec = pl.BlockSpec((1, D), lambda i: (0, 0))
  params = pltpu.CompilerParams(dimension_semantics=("arbitrary",))
  if not last:
    return pl.pallas_call(
        _tc_layer0,
        grid=(NB,),
        in_specs=[h_spec] + a_spec + cnt_spec + [w_spec, root_spec, b_spec],
        out_specs=pl.BlockSpec((BN, D), lambda i: (i, 0)),
        out_shape=jax.ShapeDtypeStruct((N, D), f32),
        compiler_params=params,
    )(hm, *([accm] * R), *([cnt] * R), w, root, b)
  batch_spec = pl.BlockSpec((1, 1, BN), lambda i: (i, 0, 0))
  return pl.pallas_call(
      _tc_layer1,
      grid=(NB,),
      in_specs=[h_spec] + a_spec + cnt_spec + [w_spec, root_spec, b_spec, batch_spec],
      out_specs=pl.BlockSpec((B, 1), lambda i: (0, 0)),
      out_shape=jax.ShapeDtypeStruct((B, 1), f32),
      scratch_shapes=[pltpu.VMEM((B, 1), f32), pltpu.VMEM((B, 1), f32)],
      compiler_params=params,
  )(hm, *([accm] * R), *([cnt] * R), w, root, b, batch2)


def kernel(atom_type, edge_index, edge_type, batch, emb, W0, root0, b0, W1, root1, b1):
  i32 = jnp.int32
  src = edge_index[0].astype(i32)
  dst = edge_index[1].astype(i32)
  et = edge_type.astype(i32)

  pad_e = E_PAD - E
  ar = jnp.arange(pad_e, dtype=i32)
  srcp = jnp.concatenate([src, ar % N])
  src4 = tuple((4 * srcp + q).reshape(NS, CPT, CHUNK) for q in range(NQ))
  dstadj2 = jnp.concatenate(
      [et * N + dst, R * N + (ar % (ACC_R - R * N))]).reshape(NS, CPT, CHUNK)
  atomp = jnp.concatenate(
      [atom_type.astype(i32), jnp.arange(NH - N, dtype=i32) % VOCAB])
  atom4 = jnp.stack([4 * atomp + q for q in range(NQ)]).reshape(NQ * 4, 20, CHUNK)
  batch2 = batch.astype(i32).reshape(NB, 1, BN)

  e4 = emb.reshape(4 * VOCAB, SLAB)

  h0m, cnt = _sc_prep_call(e4, atom4, dstadj2)
  h0v = h0m.reshape(4 * NH, SLAB)
  acc0 = _sc_edge_call(h0v, src4, dstadj2)
  h1m = _tc_layer_call(h0m, acc0, cnt, W0, root0, b0.reshape(1, D), last=False)
  h1v = h1m.reshape(4 * N, SLAB)
  acc1 = _sc_edge_call(h1v, src4, dstadj2)
  final = _tc_layer_call(h1m, acc1, cnt, W1, root1, b1.reshape(1, D),
                         last=True, batch2=batch2)
  return final[:, 0]
